# baseline (speedup 1.0000x reference)
import functools

import jax
import jax.numpy as jnp
from jax import lax
from jax.experimental import pallas as pl
from jax.experimental.pallas import tpu as pltpu
from jax.experimental.pallas import tpu_sc as plsc

_ROWS, _COLS = 8192, 4096
_NW = 32
_ROWS_PER_W = _ROWS // _NW

_MESH = plsc.VectorSubcoreMesh(core_axis_name="c", subcore_axis_name="s")


@functools.partial(
    pl.kernel,
    out_type=jax.ShapeDtypeStruct((_ROWS, _COLS), jnp.float32),
    mesh=_MESH,
)
def _sc_copy(x_hbm, o_hbm):
    wid = lax.axis_index("s") * 2 + lax.axis_index("c")
    base = wid * _ROWS_PER_W
    pltpu.sync_copy(x_hbm.at[pl.ds(base, _ROWS_PER_W)],
                    o_hbm.at[pl.ds(base, _ROWS_PER_W)])


def kernel(x):
    return _sc_copy(x)


# speedup vs baseline: 36.0752x; 36.0752x over previous
import functools

import jax
import jax.numpy as jnp
from jax import lax
from jax.experimental import pallas as pl
from jax.experimental.pallas import tpu as pltpu
from jax.experimental.pallas import tpu_sc as plsc

_ROWS, _COLS = 8192, 4096
_NW = 32
_ROWS_PER_W = _ROWS // _NW
_CHUNK_ROWS = 8
_NCHUNK = _ROWS_PER_W // _CHUNK_ROWS
_NBUF = 2

_MESH = plsc.VectorSubcoreMesh(core_axis_name="c", subcore_axis_name="s")


@functools.partial(
    pl.kernel,
    out_type=jax.ShapeDtypeStruct((_ROWS, _COLS), jnp.float32),
    mesh=_MESH,
    scratch_types=[
        pltpu.VMEM((_NBUF, _CHUNK_ROWS, _COLS), jnp.float32),
        pltpu.SemaphoreType.DMA((_NBUF,)),
        pltpu.SemaphoreType.DMA((_NBUF,)),
    ],
)
def _sc_copy(x_hbm, o_hbm, buf, in_sems, out_sems):
    wid = lax.axis_index("s") * 2 + lax.axis_index("c")
    base = wid * _ROWS_PER_W

    def chunk_slice(i):
        return pl.ds(base + i * _CHUNK_ROWS, _CHUNK_ROWS)

    for b in range(_NBUF):
        pltpu.async_copy(x_hbm.at[chunk_slice(b)], buf.at[b], in_sems.at[b])

    for i in range(_NCHUNK):
        b = i % _NBUF
        sl = chunk_slice(i)
        pltpu.make_async_copy(x_hbm.at[sl], buf.at[b], in_sems.at[b]).wait()
        pltpu.async_copy(buf.at[b], o_hbm.at[sl], out_sems.at[b])
        if i + _NBUF < _NCHUNK:
            pltpu.make_async_copy(buf.at[b], o_hbm.at[sl], out_sems.at[b]).wait()
            pltpu.async_copy(x_hbm.at[chunk_slice(i + _NBUF)], buf.at[b],
                             in_sems.at[b])

    for i in range(_NCHUNK - _NBUF, _NCHUNK):
        b = i % _NBUF
        pltpu.make_async_copy(buf.at[b], o_hbm.at[chunk_slice(i)],
                              out_sems.at[b]).wait()


def kernel(x):
    return _sc_copy(x)
